# baseline (device time: 153316 ns/iter reference)
import jax
import jax.numpy as jnp
from jax import lax
from jax.experimental import pallas as pl
from jax.experimental.pallas import tpu as pltpu

N_Z = 4
SUB = 2


def kernel(O, Wo):
    B, S, Hs, D = O.shape
    K = Hs * D
    N = Wo.shape[1]
    s_per = S // N_Z
    n_hops = N_Z - 1
    n_sub = B // SUB

    x = O.reshape(B, S, K)

    def body(x_ref, w_ref, out_ref, comm_ref, wb_ref, send_sems, recv_sems):
        my_x = lax.axis_index("x")
        my_y = lax.axis_index("y")
        my_z = lax.axis_index("z")
        left = (my_z - 1) % N_Z
        right = (my_z + 1) % N_Z

        barrier_sem = pltpu.get_barrier_semaphore()
        for nbr in (left, right):
            pl.semaphore_signal(
                barrier_sem, inc=1,
                device_id=(my_x, my_y, nbr),
                device_id_type=pl.DeviceIdType.MESH,
            )
        pl.semaphore_wait(barrier_sem, 2)

        wb_ref[:, :] = w_ref[:, :].astype(jnp.bfloat16)

        def chunk_f32(c, b):
            xs = x_ref[b, pl.ds(c * s_per, s_per), :].astype(jnp.bfloat16)
            return lax.dot_general(
                xs, wb_ref[:, :],
                (((1,), (0,)), ((), ())),
                preferred_element_type=jnp.float32,
            )

        rdmas = [
            [
                pltpu.make_async_remote_copy(
                    src_ref=comm_ref.at[h, g],
                    dst_ref=comm_ref.at[h + 1, g],
                    send_sem=send_sems.at[h, g],
                    recv_sem=recv_sems.at[h, g],
                    device_id=(my_x, my_y, right),
                    device_id_type=pl.DeviceIdType.MESH,
                )
                for g in range(n_sub)
            ]
            for h in range(n_hops)
        ]

        c0 = (my_z - 1) % N_Z
        for g in range(n_sub):
            for i in range(SUB):
                comm_ref[0, g, pl.ds(i * s_per, s_per), :] = (
                    chunk_f32(c0, g * SUB + i).astype(jnp.bfloat16)
                )
            rdmas[0][g].start()

        for h in range(n_hops):
            c = (my_z - 2 - h) % N_Z
            for g in range(n_sub):
                if h < n_hops - 1:
                    ts = [
                        chunk_f32(c, g * SUB + i).astype(jnp.bfloat16)
                        for i in range(SUB)
                    ]
                    rdmas[h][g].wait()
                    for i in range(SUB):
                        sl = pl.ds(i * s_per, s_per)
                        comm_ref[h + 1, g, sl, :] = (
                            comm_ref[h + 1, g, sl, :] + ts[i]
                        )
                    rdmas[h + 1][g].start()
                else:
                    ts = [chunk_f32(c, g * SUB + i) for i in range(SUB)]
                    rdmas[h][g].wait()
                    for i in range(SUB):
                        sl = pl.ds(i * s_per, s_per)
                        out_ref[g * SUB + i, :, :] = (
                            ts[i]
                            + comm_ref[h + 1, g, sl, :].astype(jnp.float32)
                        )

    return pl.pallas_call(
        body,
        out_shape=jax.ShapeDtypeStruct((B, s_per, N), jnp.float32),
        in_specs=[
            pl.BlockSpec(memory_space=pltpu.VMEM),
            pl.BlockSpec(memory_space=pltpu.VMEM),
        ],
        out_specs=pl.BlockSpec(memory_space=pltpu.VMEM),
        scratch_shapes=[
            pltpu.VMEM((N_Z, n_sub, SUB * s_per, N), jnp.bfloat16),
            pltpu.VMEM((K, N), jnp.bfloat16),
            pltpu.SemaphoreType.DMA((N_Z - 1, n_sub)),
            pltpu.SemaphoreType.DMA((N_Z - 1, n_sub)),
        ],
        compiler_params=pltpu.CompilerParams(collective_id=0),
    )(x, Wo)
